# Initial kernel scaffold; baseline (speedup 1.0000x reference)
import jax
import jax.numpy as jnp
from jax.experimental import pallas as pl


def kernel(x, w_gate, W1, b1, W2, b2):
    raise NotImplementedError("write your pallas kernel here")


# speedup vs baseline: 2.1834x; 2.1834x over previous
import functools

import jax
import jax.numpy as jnp
import numpy as np
from jax.experimental import pallas as pl
from jax.experimental.pallas import tpu as pltpu

E = 16
TOPK = 2
D_IN = 768
D_HID = 256
D_OUT = 768
B = 2048
LOSS_COEF = 0.01
_EPS = float(np.finfo(float).eps)

_BB = 256
_XB = 512


def _router_body(x_ref, wg_ref, gates_ref, imp_ref, load_ref):
    i = pl.program_id(0)
    logits = jnp.dot(x_ref[...], wg_ref[...], preferred_element_type=jnp.float32)
    idx = jax.lax.broadcasted_iota(jnp.int32, logits.shape, 1)
    m1 = jnp.max(logits, axis=1, keepdims=True)
    a1 = jnp.min(jnp.where(logits == m1, idx, E), axis=1, keepdims=True)
    oh1 = idx == a1
    masked = jnp.where(oh1, -jnp.inf, logits)
    m2 = jnp.max(masked, axis=1, keepdims=True)
    a2 = jnp.min(jnp.where(masked == m2, idx, E), axis=1, keepdims=True)
    oh2 = idx == a2
    t = jnp.exp(m2 - m1)
    g1 = 1.0 / (1.0 + t)
    g2 = t / (1.0 + t)
    gates = jnp.where(oh1, g1, 0.0) + jnp.where(oh2, g2, 0.0)
    gates_ref[...] = gates

    @pl.when(i == 0)
    def _():
        imp_ref[...] = jnp.zeros_like(imp_ref)
        load_ref[...] = jnp.zeros_like(load_ref)

    imp_ref[...] += jnp.sum(gates, axis=0, keepdims=True)
    load_ref[...] += jnp.sum((gates > 0).astype(jnp.float32), axis=0, keepdims=True)


def _expert_body(x_ref, w1_ref, b1_ref, w2_ref, b2_ref, gates_ref, out_ref):
    e = pl.program_id(1)
    x = x_ref[...]
    h = jnp.maximum(
        jnp.dot(x, w1_ref[0], preferred_element_type=jnp.float32) + b1_ref[0],
        0.0,
    )
    o = jnp.dot(h, w2_ref[0], preferred_element_type=jnp.float32) + b2_ref[0]
    m = jnp.max(o, axis=-1, keepdims=True)
    p = jnp.exp(o - m)
    sm = p / jnp.sum(p, axis=-1, keepdims=True)
    lane = jax.lax.broadcasted_iota(jnp.int32, gates_ref.shape, 1)
    gcol = jnp.sum(jnp.where(lane == e, gates_ref[...], 0.0), axis=1)

    @pl.when(e == 0)
    def _():
        out_ref[...] = jnp.zeros_like(out_ref)

    out_ref[...] += gcol[:, None] * sm

    @pl.when(e == E - 1)
    def _():
        acc = out_ref[...]
        out_ref[...] = jnp.log(jnp.where(acc == 0.0, _EPS, acc))


def _cv_sq(v):
    eps = 1e-10
    return jnp.var(v, ddof=1) / (jnp.mean(v) ** 2 + eps)


@jax.jit
def kernel(x, w_gate, W1, b1, W2, b2):
    nb = B // _BB
    gates, imp, load = pl.pallas_call(
        _router_body,
        grid=(nb,),
        in_specs=[
            pl.BlockSpec((_BB, D_IN), lambda i: (i, 0)),
            pl.BlockSpec((D_IN, E), lambda i: (0, 0)),
        ],
        out_specs=[
            pl.BlockSpec((_BB, E), lambda i: (i, 0)),
            pl.BlockSpec((1, E), lambda i: (0, 0)),
            pl.BlockSpec((1, E), lambda i: (0, 0)),
        ],
        out_shape=[
            jax.ShapeDtypeStruct((B, E), jnp.float32),
            jax.ShapeDtypeStruct((1, E), jnp.float32),
            jax.ShapeDtypeStruct((1, E), jnp.float32),
        ],
    )(x, w_gate)

    nxb = B // _XB
    y = pl.pallas_call(
        _expert_body,
        grid=(nxb, E),
        in_specs=[
            pl.BlockSpec((_XB, D_IN), lambda i, e: (i, 0)),
            pl.BlockSpec((1, D_IN, D_HID), lambda i, e: (e, 0, 0)),
            pl.BlockSpec((1, 1, D_HID), lambda i, e: (e, 0, 0)),
            pl.BlockSpec((1, D_HID, D_OUT), lambda i, e: (e, 0, 0)),
            pl.BlockSpec((1, 1, D_OUT), lambda i, e: (e, 0, 0)),
            pl.BlockSpec((_XB, E), lambda i, e: (i, 0)),
        ],
        out_specs=pl.BlockSpec((_XB, D_OUT), lambda i, e: (i, 0)),
        out_shape=jax.ShapeDtypeStruct((B, D_OUT), jnp.float32),
        compiler_params=pltpu.CompilerParams(
            dimension_semantics=("arbitrary", "arbitrary"),
        ),
    )(x, W1, b1[:, None, :], W2, b2[:, None, :], gates)

    loss = (_cv_sq(imp[0]) + _cv_sq(load[0])) * LOSS_COEF
    return y, loss
